# Initial kernel scaffold; baseline (speedup 1.0000x reference)
import jax
import jax.numpy as jnp
from jax.experimental import pallas as pl


def kernel(x):
    raise NotImplementedError("write your pallas kernel here")


# speedup vs baseline: 1.9442x; 1.9442x over previous
import functools

import jax
import jax.numpy as jnp
from jax import lax
from jax.experimental import pallas as pl

_ROWS_PER_STEP = 8


def _fused_body(keep_k, x_ref, pruned_ref, mask_ref):
    xb = x_ref[...]
    tokens = jnp.mean(jnp.abs(xb), axis=1)
    tbits = lax.bitcast_convert_type(tokens, jnp.int32)

    def bs_step(_, carry):
        lo, hi = carry
        mid = lo + ((hi - lo + 1) >> 1)
        cnt = jnp.sum((tbits >= mid).astype(jnp.int32), axis=1, keepdims=True)
        ge = cnt >= keep_k
        return jnp.where(ge, mid, lo), jnp.where(ge, hi, mid - 1)

    r = tokens.shape[0]
    lo0 = jnp.zeros((r, 1), jnp.int32)
    hi0 = jnp.full((r, 1), 0x7F800000, jnp.int32)
    thr, _ = lax.fori_loop(0, 31, bs_step, (lo0, hi0))

    gt = tbits > thr
    eq = tbits == thr
    need = keep_k - jnp.sum(gt.astype(jnp.int32), axis=1, keepdims=True)

    hw = tokens.shape[1]
    col = lax.broadcasted_iota(jnp.int32, tokens.shape, 1)
    eq_i = eq.astype(jnp.int32)

    def tie_step(_, carry):
        lo, hi = carry
        mid = (lo + hi) >> 1
        cnt = jnp.sum(eq_i * (col < mid).astype(jnp.int32), axis=1, keepdims=True)
        ge = cnt >= need
        return jnp.where(ge, lo, mid), jnp.where(ge, mid, hi)

    nbits = max(1, (hw).bit_length())
    lo0 = jnp.zeros((r, 1), jnp.int32)
    hi0 = jnp.full((r, 1), hw, jnp.int32)
    _, cutoff = lax.fori_loop(0, nbits, tie_step, (lo0, hi0))
    keep = gt | (eq & (col < cutoff))
    mask_ref[...] = keep.astype(jnp.int32)
    pruned_ref[...] = xb * keep[:, None, :].astype(xb.dtype)


def kernel(x):
    B, T, C, H, W = x.shape
    BT, HW = B * T, H * W
    keep_k = max(1, int(HW * 0.5))
    R = _ROWS_PER_STEP
    x3 = x.reshape(BT, C, HW)

    pruned3, mask_i = pl.pallas_call(
        functools.partial(_fused_body, keep_k),
        grid=(BT // R,),
        in_specs=[pl.BlockSpec((R, C, HW), lambda i: (i, 0, 0))],
        out_specs=[
            pl.BlockSpec((R, C, HW), lambda i: (i, 0, 0)),
            pl.BlockSpec((R, HW), lambda i: (i, 0)),
        ],
        out_shape=[
            jax.ShapeDtypeStruct((BT, C, HW), x.dtype),
            jax.ShapeDtypeStruct((BT, HW), jnp.int32),
        ],
    )(x3)

    pruned = pruned3.reshape(B, T, C, H, W)
    mask_2d = mask_i.astype(bool).reshape(B, T, H, W)
    return (pruned, mask_2d, mask_2d)


# speedup vs baseline: 2.2206x; 1.1421x over previous
import functools

import jax
import jax.numpy as jnp
from jax import lax
from jax.experimental import pallas as pl

_ROWS_PER_STEP = 16


def _fused_body(keep_k, x_ref, pruned_ref, mask_ref):
    xb = x_ref[...]
    tokens = jnp.mean(jnp.abs(xb), axis=1)
    tbits = lax.bitcast_convert_type(tokens, jnp.int32)

    def bs_step(_, carry):
        lo, hi = carry
        mid = lo + ((hi - lo + 1) >> 1)
        cnt = jnp.sum((tbits >= mid).astype(jnp.int32), axis=1, keepdims=True)
        ge = cnt >= keep_k
        return jnp.where(ge, mid, lo), jnp.where(ge, hi, mid - 1)

    r = tokens.shape[0]
    lo0 = jnp.zeros((r, 1), jnp.int32)
    hi0 = jnp.full((r, 1), 0x7F800000, jnp.int32)
    thr, _ = lax.fori_loop(0, 31, bs_step, (lo0, hi0))

    gt = tbits > thr
    eq = tbits == thr
    need = keep_k - jnp.sum(gt.astype(jnp.int32), axis=1, keepdims=True)

    hw = tokens.shape[1]
    col = lax.broadcasted_iota(jnp.int32, tokens.shape, 1)
    eq_i = eq.astype(jnp.int32)

    def tie_step(_, carry):
        lo, hi = carry
        mid = (lo + hi) >> 1
        cnt = jnp.sum(eq_i * (col < mid).astype(jnp.int32), axis=1, keepdims=True)
        ge = cnt >= need
        return jnp.where(ge, lo, mid), jnp.where(ge, mid, hi)

    nbits = max(1, (hw).bit_length())
    lo0 = jnp.zeros((r, 1), jnp.int32)
    hi0 = jnp.full((r, 1), hw, jnp.int32)
    _, cutoff = lax.fori_loop(0, nbits, tie_step, (lo0, hi0))
    keep = gt | (eq & (col < cutoff))
    mask_ref[...] = keep.astype(jnp.int32)
    pruned_ref[...] = xb * keep[:, None, :].astype(xb.dtype)


def kernel(x):
    B, T, C, H, W = x.shape
    BT, HW = B * T, H * W
    keep_k = max(1, int(HW * 0.5))
    R = _ROWS_PER_STEP
    x3 = x.reshape(BT, C, HW)

    pruned3, mask_i = pl.pallas_call(
        functools.partial(_fused_body, keep_k),
        grid=(BT // R,),
        in_specs=[pl.BlockSpec((R, C, HW), lambda i: (i, 0, 0))],
        out_specs=[
            pl.BlockSpec((R, C, HW), lambda i: (i, 0, 0)),
            pl.BlockSpec((R, HW), lambda i: (i, 0)),
        ],
        out_shape=[
            jax.ShapeDtypeStruct((BT, C, HW), x.dtype),
            jax.ShapeDtypeStruct((BT, HW), jnp.int32),
        ],
    )(x3)

    pruned = pruned3.reshape(B, T, C, H, W)
    mask_2d = mask_i.astype(bool).reshape(B, T, H, W)
    return (pruned, mask_2d, mask_2d)
